# Initial kernel scaffold; baseline (speedup 1.0000x reference)
import jax
import jax.numpy as jnp
from jax.experimental import pallas as pl


def kernel(x, edge_index, community, id_embeddings, W_emb1, b_emb1, W_emb2, b_emb2, W_emb3, b_emb3, W_conv1, b_conv1, W_conv2, b_conv2, W_lin1, b_lin1, W_lin2, b_lin2):
    raise NotImplementedError("write your pallas kernel here")


# speedup vs baseline: 15.8287x; 15.8287x over previous
import functools

import jax
import jax.numpy as jnp
from jax import lax
from jax.experimental import pallas as pl
from jax.experimental.pallas import tpu as pltpu
from jax.experimental.pallas import tpu_sc as plsc

N = 50000
E = 800000
NCOMM = 1000
NID = 32
NH = 64

NP = 53248
EP = 819200
CP = 1024

NPR = NP // 128
EPR = EP // 128

SC_CORES = 2
SC_TILES = 16

_sc_mesh = plsc.VectorSubcoreMesh(
    core_axis_name="c", subcore_axis_name="s",
    num_cores=SC_CORES, num_subcores=SC_TILES)

f32 = jnp.float32


def _fill_ones(ref, n):
    for i in range(n // 16):
        ref[pl.ds(i * 16, 16)] = jnp.full((16,), 1.0, f32)


@functools.partial(
    pl.kernel,
    out_type=(
        jax.ShapeDtypeStruct((SC_CORES * NP,), f32),
        jax.ShapeDtypeStruct((SC_CORES * CP,), f32),
    ),
    mesh=_sc_mesh,
    compiler_params=pltpu.CompilerParams(use_tc_tiling_on_sc=False),
    scratch_types=[
        pltpu.VMEM_SHARED((NP,), f32),
        pltpu.VMEM_SHARED((CP,), f32),
        pltpu.VMEM((EPR // 32, 128), jnp.int32),
        pltpu.VMEM((16, 128), jnp.int32),
        pltpu.VMEM((128,), f32),
    ],
)
def _counts_kernel(dst2d, comm2d, zdeg, zcnt, deg_out, cnt_out,
                   acc_deg, acc_cnt, idxd, idxc, ones_v):
    c = lax.axis_index("c")
    s = lax.axis_index("s")
    w = c * SC_TILES + s
    dchunk = NP // SC_TILES
    pltpu.sync_copy(zdeg.at[pl.ds(s * dchunk, dchunk)],
                    acc_deg.at[pl.ds(s * dchunk, dchunk)])

    @pl.when(s == 0)
    def _():
        pltpu.sync_copy(zcnt, acc_cnt)

    _fill_ones(ones_v, 128)
    nd = EPR // 32
    pltpu.sync_copy(dst2d.at[pl.ds(c * (EPR // 2) + s * nd, nd)], idxd)
    nchunks = NPR // 16

    @pl.when(w < nchunks)
    def _():
        pltpu.sync_copy(comm2d.at[pl.ds(w * 16, 16)], idxc)

    plsc.subcore_barrier()

    @pl.loop(0, nd)
    def _(j):
        pltpu.sync_copy(ones_v, acc_deg.at[idxd.at[j]], add=True)

    @pl.when(w < nchunks)
    def _():
        @pl.loop(0, 16)
        def _(j):
            pltpu.sync_copy(ones_v, acc_cnt.at[idxc.at[j]], add=True)

    plsc.subcore_barrier()
    pltpu.sync_copy(acc_deg.at[pl.ds(s * dchunk, dchunk)],
                    deg_out.at[pl.ds(c * NP + s * dchunk, dchunk)])
    @pl.when(s < 8)
    def _():
        pltpu.sync_copy(acc_cnt.at[pl.ds(s * 128, 128)],
                        cnt_out.at[pl.ds(c * CP + s * 128, 128)])


@functools.partial(
    pl.kernel,
    out_type=jax.ShapeDtypeStruct((SC_CORES * NP, 32), f32),
    mesh=_sc_mesh,
    compiler_params=pltpu.CompilerParams(use_tc_tiling_on_sc=False),
    scratch_types=[
        pltpu.VMEM_SHARED((NP, 32), f32),
        pltpu.VMEM((40, 128), jnp.int32),
        pltpu.VMEM((40, 128), jnp.int32),
        pltpu.VMEM((256, 32), f32),
        pltpu.SemaphoreType.DMA,
        pltpu.SemaphoreType.DMA,
    ],
)
def _agg_kernel(gflat, src2d, dst2d, out, acc, srcv, dstv, rows, sem0, sem1):
    c = lax.axis_index("c")
    s = lax.axis_index("s")
    rchunk = NP // SC_TILES
    pltpu.sync_copy(gflat.at[pl.ds(c * NP + s * rchunk, rchunk)],
                    acc.at[pl.ds(s * rchunk, rchunk)])
    plsc.subcore_barrier()
    nrows = EPR // SC_TILES
    nchunk = 40

    @pl.loop(0, nrows // nchunk)
    def _(k):
        base = s * nrows + k * nchunk
        pltpu.sync_copy(src2d.at[pl.ds(c * EPR + base, nchunk)], srcv)
        pltpu.sync_copy(dst2d.at[pl.ds(base, nchunk)], dstv)
        pltpu.async_copy(gflat.at[srcv.at[0]], rows.at[pl.ds(0, 128)], sem0)

        @pl.loop(0, nchunk // 2)
        def _(p):
            j0 = 2 * p
            pltpu.async_copy(gflat.at[srcv.at[j0 + 1]],
                             rows.at[pl.ds(128, 128)], sem1)
            pltpu.make_async_copy(gflat.at[srcv.at[j0]],
                                  rows.at[pl.ds(0, 128)], sem0).wait()
            pltpu.sync_copy(rows.at[pl.ds(0, 128)], acc.at[dstv.at[j0]],
                            add=True)

            @pl.when(p < nchunk // 2 - 1)
            def _():
                pltpu.async_copy(gflat.at[srcv.at[j0 + 2]],
                                 rows.at[pl.ds(0, 128)], sem0)

            pltpu.make_async_copy(gflat.at[srcv.at[j0 + 1]],
                                  rows.at[pl.ds(128, 128)], sem1).wait()
            pltpu.sync_copy(rows.at[pl.ds(128, 128)], acc.at[dstv.at[j0 + 1]],
                            add=True)

    plsc.subcore_barrier()
    pltpu.sync_copy(acc.at[pl.ds(s * rchunk, rchunk)],
                    out.at[pl.ds(c * NP + s * rchunk, rchunk)])


@functools.partial(
    pl.kernel,
    out_type=jax.ShapeDtypeStruct((SC_CORES * CP, NH), f32),
    mesh=_sc_mesh,
    compiler_params=pltpu.CompilerParams(use_tc_tiling_on_sc=False),
    scratch_types=[
        pltpu.VMEM_SHARED((CP, NH), f32),
        pltpu.VMEM((128, NH), f32),
        pltpu.VMEM((16, 128), jnp.int32),
    ],
)
def _pool_kernel(h, comm2d, zpool, out, accp, hv, idxc):
    c = lax.axis_index("c")
    s = lax.axis_index("s")
    w = c * SC_TILES + s
    pchunk = CP // SC_TILES
    pltpu.sync_copy(zpool.at[pl.ds(s * pchunk, pchunk)],
                    accp.at[pl.ds(s * pchunk, pchunk)])
    nchunks = NPR // 16

    @pl.when(w < nchunks)
    def _():
        pltpu.sync_copy(comm2d.at[pl.ds(w * 16, 16)], idxc)

    plsc.subcore_barrier()

    @pl.when(w < nchunks)
    def _():
        base = w * 16 * 128

        @pl.loop(0, 16)
        def _(j):
            pltpu.sync_copy(h.at[pl.ds(base + j * 128, 128)], hv)
            pltpu.sync_copy(hv, accp.at[idxc.at[j]], add=True)

    plsc.subcore_barrier()
    pltpu.sync_copy(accp.at[pl.ds(s * pchunk, pchunk)],
                    out.at[pl.ds(c * CP + s * pchunk, pchunk)])


_BLK = 512
_NBLK = NP // _BLK


def _dinv_of(degp):
    deg = degp[0] + degp[1] + 1.0
    return lax.rsqrt(deg)


def _dense_body(x_ref, id_ref, degp_ref, w1, b1, w2, b2, w3, b3, wc1, out_ref):
    xb = x_ref[...]
    x1 = jnp.maximum(jnp.dot(xb[:, :7], w1[...],
                             preferred_element_type=f32) + b1[...], 0.0)
    x2 = jnp.maximum(jnp.dot(xb[:, 7:], w2[...],
                             preferred_element_type=f32) + b2[...], 0.0)
    hcat = jnp.concatenate([x1, x2, id_ref[...]], axis=1)
    h = jnp.maximum(jnp.dot(hcat, w3[...],
                            preferred_element_type=f32) + b3[...], 0.0)
    dinv = _dinv_of(degp_ref[...])
    g = jnp.dot(h, wc1[0], preferred_element_type=f32) * dinv[:, None]
    out_ref[...] = g[None]


def _dense_call(xp, idp, deg2, w1, b1, w2, b2, w3, b3, wc1):
    return pl.pallas_call(
        _dense_body,
        grid=(2, _NBLK),
        in_specs=[
            pl.BlockSpec((_BLK, 19), lambda j, i: (i, 0)),
            pl.BlockSpec((_BLK, NID), lambda j, i: (i, 0)),
            pl.BlockSpec((2, _BLK), lambda j, i: (0, i)),
            pl.BlockSpec((7, 32), lambda j, i: (0, 0)),
            pl.BlockSpec((1, 32), lambda j, i: (0, 0)),
            pl.BlockSpec((12, 32), lambda j, i: (0, 0)),
            pl.BlockSpec((1, 32), lambda j, i: (0, 0)),
            pl.BlockSpec((96, 96), lambda j, i: (0, 0)),
            pl.BlockSpec((1, 96), lambda j, i: (0, 0)),
            pl.BlockSpec((1, 96, 32), lambda j, i: (j, 0, 0)),
        ],
        out_specs=pl.BlockSpec((1, _BLK, 32), lambda j, i: (j, i, 0)),
        out_shape=jax.ShapeDtypeStruct((2, NP, 32), f32),
    )(xp, idp, deg2, w1, b1, w2, b2, w3, b3, wc1)


def _post1_body(agg_ref, degp_ref, bc1, wc2, h1_ref, g2_ref):
    ab = agg_ref[...]
    a = jnp.concatenate([ab[0], ab[1]], axis=1)
    dinv = _dinv_of(degp_ref[...])
    h1 = jnp.maximum(a * dinv[:, None] + bc1[...], 0.0)
    h1_ref[...] = h1
    g2_ref[...] = (jnp.dot(h1, wc2[0], preferred_element_type=f32)
                   * dinv[:, None])[None]


def _post1_call(agg1, deg2, bc1, wc2):
    return pl.pallas_call(
        _post1_body,
        grid=(2, _NBLK),
        in_specs=[
            pl.BlockSpec((2, _BLK, 32), lambda j, i: (0, i, 0)),
            pl.BlockSpec((2, _BLK), lambda j, i: (0, i)),
            pl.BlockSpec((1, NH), lambda j, i: (0, 0)),
            pl.BlockSpec((1, NH, 32), lambda j, i: (j, 0, 0)),
        ],
        out_specs=[
            pl.BlockSpec((_BLK, NH), lambda j, i: (i, 0)),
            pl.BlockSpec((1, _BLK, 32), lambda j, i: (j, i, 0)),
        ],
        out_shape=[
            jax.ShapeDtypeStruct((NP, NH), f32),
            jax.ShapeDtypeStruct((2, NP, 32), f32),
        ],
    )(agg1, deg2, bc1, wc2)


def _post2_body(agg_ref, degp_ref, bc2, h2_ref):
    ab = agg_ref[...]
    a = jnp.concatenate([ab[0], ab[1]], axis=1)
    dinv = _dinv_of(degp_ref[...])
    h2_ref[...] = jnp.maximum(a * dinv[:, None] + bc2[...], 0.0)


def _post2_call(agg2, deg2, bc2):
    return pl.pallas_call(
        _post2_body,
        grid=(_NBLK,),
        in_specs=[
            pl.BlockSpec((2, _BLK, 32), lambda i: (0, i, 0)),
            pl.BlockSpec((2, _BLK), lambda i: (0, i)),
            pl.BlockSpec((1, NH), lambda i: (0, 0)),
        ],
        out_specs=pl.BlockSpec((_BLK, NH), lambda i: (i, 0)),
        out_shape=jax.ShapeDtypeStruct((NP, NH), f32),
    )(agg2, deg2, bc2)


def _final_body(p1_ref, p2_ref, cnt_ref, wl1, bl1, wl2, bl2, out_ref):
    p1 = p1_ref[...]
    p2 = p2_ref[...]
    ssum = p1[:CP] + p1[CP:] + p2[:CP] + p2[CP:]
    cntp = cnt_ref[...]
    cnt = jnp.maximum(cntp[0] + cntp[1], 1.0)
    r = ssum / cnt[:, None]
    w = wl1[...]
    wf = w[:NH] + w[NH:]
    z = jnp.maximum(jnp.dot(r, wf, preferred_element_type=f32) + bl1[...], 0.0)
    out_ref[...] = jnp.dot(z, wl2[...], preferred_element_type=f32) + bl2[...]


def _final_call(pools1, pools2, cnt2, wl1, bl1, wl2, bl2):
    return pl.pallas_call(
        _final_body,
        out_shape=jax.ShapeDtypeStruct((CP, 1), f32),
    )(pools1, pools2, cnt2, wl1, bl1, wl2, bl2)


def kernel(x, edge_index, community, id_embeddings, W_emb1, b_emb1, W_emb2,
           b_emb2, W_emb3, b_emb3, W_conv1, b_conv1, W_conv2, b_conv2,
           W_lin1, b_lin1, W_lin2, b_lin2):
    src = edge_index[0]
    dst = edge_index[1]
    epad = jnp.full((EP - E,), N, jnp.int32)
    srcp = jnp.concatenate([src, epad])
    dst2d = jnp.concatenate([dst, epad]).reshape(EPR, 128)
    src2d = jnp.concatenate([srcp, srcp + NP]).reshape(2 * EPR, 128)
    comm2d = jnp.concatenate(
        [community, jnp.full((NP - N,), NCOMM, jnp.int32)]).reshape(NPR, 128)
    xp = jnp.concatenate([x, jnp.zeros((NP - N, x.shape[1]), f32)])
    idp = jnp.concatenate([id_embeddings, jnp.zeros((NP - N, NID), f32)])
    zdeg = jnp.zeros((NP,), f32)
    zcnt = jnp.zeros((CP,), f32)
    zpool = jnp.zeros((CP, NH), f32)

    deg_part, cnt_part = _counts_kernel(dst2d, comm2d, zdeg, zcnt)
    deg2 = deg_part.reshape(2, NP)

    wc1_2 = W_conv1.reshape(96, 2, 32).transpose(1, 0, 2)
    wc2_2 = W_conv2.reshape(NH, 2, 32).transpose(1, 0, 2)
    g1 = _dense_call(xp, idp, deg2, W_emb1, b_emb1.reshape(1, -1), W_emb2,
                     b_emb2.reshape(1, -1), W_emb3, b_emb3.reshape(1, -1),
                     wc1_2)
    agg1 = _agg_kernel(g1.reshape(2 * NP, 32), src2d, dst2d)
    h1, g2 = _post1_call(agg1.reshape(2, NP, 32), deg2,
                         b_conv1.reshape(1, -1), wc2_2)
    pools1 = _pool_kernel(h1, comm2d, zpool)
    agg2 = _agg_kernel(g2.reshape(2 * NP, 32), src2d, dst2d)
    h2 = _post2_call(agg2.reshape(2, NP, 32), deg2, b_conv2.reshape(1, -1))
    pools2 = _pool_kernel(h2, comm2d, zpool)

    res = _final_call(pools1, pools2, cnt_part.reshape(2, CP), W_lin1,
                      b_lin1.reshape(1, -1), W_lin2, b_lin2.reshape(1, 1))
    return res[:NCOMM, 0]


# speedup vs baseline: 17.0342x; 1.0762x over previous
import functools

import jax
import jax.numpy as jnp
from jax import lax
from jax.experimental import pallas as pl
from jax.experimental.pallas import tpu as pltpu
from jax.experimental.pallas import tpu_sc as plsc

N = 50000
E = 800000
NCOMM = 1000
NID = 32
NH = 64

NP = 53248
EP = 819200
CP = 1024

NPR = NP // 128
EPR = EP // 128

SC_CORES = 2
SC_TILES = 16

_sc_mesh = plsc.VectorSubcoreMesh(
    core_axis_name="c", subcore_axis_name="s",
    num_cores=SC_CORES, num_subcores=SC_TILES)

f32 = jnp.float32


def _fill_ones(ref, n):
    for i in range(n // 16):
        ref[pl.ds(i * 16, 16)] = jnp.full((16,), 1.0, f32)


@functools.partial(
    pl.kernel,
    out_type=(
        jax.ShapeDtypeStruct((SC_CORES * NP,), f32),
        jax.ShapeDtypeStruct((SC_CORES * CP,), f32),
    ),
    mesh=_sc_mesh,
    compiler_params=pltpu.CompilerParams(use_tc_tiling_on_sc=False),
    scratch_types=[
        pltpu.VMEM_SHARED((NP,), f32),
        pltpu.VMEM_SHARED((CP,), f32),
        pltpu.VMEM((EPR // 32, 128), jnp.int32),
        pltpu.VMEM((16, 128), jnp.int32),
        pltpu.VMEM((128,), f32),
    ],
)
def _counts_kernel(dst2d, comm2d, zdeg, zcnt, deg_out, cnt_out,
                   acc_deg, acc_cnt, idxd, idxc, ones_v):
    c = lax.axis_index("c")
    s = lax.axis_index("s")
    w = c * SC_TILES + s
    dchunk = NP // SC_TILES
    pltpu.sync_copy(zdeg.at[pl.ds(s * dchunk, dchunk)],
                    acc_deg.at[pl.ds(s * dchunk, dchunk)])

    @pl.when(s == 0)
    def _():
        pltpu.sync_copy(zcnt, acc_cnt)

    _fill_ones(ones_v, 128)
    nd = EPR // 32
    pltpu.sync_copy(dst2d.at[pl.ds(c * (EPR // 2) + s * nd, nd)], idxd)
    nchunks = NPR // 16

    @pl.when(w < nchunks)
    def _():
        pltpu.sync_copy(comm2d.at[pl.ds(w * 16, 16)], idxc)

    plsc.subcore_barrier()

    @pl.loop(0, nd)
    def _(j):
        pltpu.sync_copy(ones_v, acc_deg.at[idxd.at[j]], add=True)

    @pl.when(w < nchunks)
    def _():
        @pl.loop(0, 16)
        def _(j):
            pltpu.sync_copy(ones_v, acc_cnt.at[idxc.at[j]], add=True)

    plsc.subcore_barrier()
    pltpu.sync_copy(acc_deg.at[pl.ds(s * dchunk, dchunk)],
                    deg_out.at[pl.ds(c * NP + s * dchunk, dchunk)])
    @pl.when(s < 8)
    def _():
        pltpu.sync_copy(acc_cnt.at[pl.ds(s * 128, 128)],
                        cnt_out.at[pl.ds(c * CP + s * 128, 128)])


@functools.partial(
    pl.kernel,
    out_type=jax.ShapeDtypeStruct((SC_CORES * NP, 32), f32),
    mesh=_sc_mesh,
    compiler_params=pltpu.CompilerParams(use_tc_tiling_on_sc=False),
    scratch_types=[
        pltpu.VMEM_SHARED((NP, 32), f32),
        pltpu.VMEM((2, 8, 128), jnp.int32),
        pltpu.VMEM((2, 8, 128), jnp.int32),
        pltpu.VMEM((512, 32), f32),
        pltpu.SemaphoreType.DMA,
        pltpu.SemaphoreType.DMA,
        pltpu.SemaphoreType.DMA,
        pltpu.SemaphoreType.DMA,
        pltpu.SemaphoreType.DMA,
        pltpu.SemaphoreType.DMA,
        pltpu.SemaphoreType.DMA,
        pltpu.SemaphoreType.DMA,
        pltpu.SemaphoreType.DMA,
    ],
)
def _agg_kernel(gflat, src2d, dst2d, out, acc, srcv, dstv, rows,
                g0, g1, g2, g3, s0, s1, s2, s3, isem):
    c = lax.axis_index("c")
    s = lax.axis_index("s")
    gsem = (g0, g1, g2, g3)
    ssem = (s0, s1, s2, s3)
    rchunk = NP // SC_TILES
    pltpu.sync_copy(gflat.at[pl.ds(c * NP + s * rchunk, rchunk)],
                    acc.at[pl.ds(s * rchunk, rchunk)])
    plsc.subcore_barrier()
    nrows = EPR // SC_TILES
    base = s * nrows
    nch = nrows // 8

    def idx_copies(k, half):
        a = pltpu.make_async_copy(
            src2d.at[pl.ds(c * EPR + base + k * 8, 8)], srcv.at[half], isem)
        d = pltpu.make_async_copy(
            dst2d.at[pl.ds(base + k * 8, 8)], dstv.at[half], isem)
        return a, d

    def slot(b):
        return rows.at[pl.ds((b % 4) * 128, 128)]

    def fire_gather(hk, r, b):
        pltpu.async_copy(gflat.at[srcv.at[hk, r]], slot(b), gsem[b % 4])

    def wait_gather(hk, r, b):
        pltpu.make_async_copy(gflat.at[srcv.at[hk, r]], slot(b),
                              gsem[b % 4]).wait()

    def fire_scatter(hk, r, b):
        pltpu.async_copy(slot(b), acc.at[dstv.at[hk, r]], ssem[b % 4],
                         add=True)

    def wait_scatter(hk, r, b):
        pltpu.make_async_copy(slot(b), acc.at[dstv.at[hk, r]],
                              ssem[b % 4]).wait()

    a, d = idx_copies(0, 0)
    a.start()
    d.start()
    a.wait()
    d.wait()

    @pl.loop(0, nch)
    def _(k):
        hk = k % 2

        @pl.when(k > 0)
        def _():
            a, d = idx_copies(k, hk)
            a.wait()
            d.wait()

        for r in range(8):
            def wait_prev_scatter():
                if r >= 4:
                    wait_scatter(hk, r - 4, r - 4)
                else:
                    wait_scatter(1 - hk, r + 4, r)

            if r >= 4:
                wait_prev_scatter()
            else:
                pl.when(k > 0)(wait_prev_scatter)

            fire_gather(hk, r, r)

            if r == 4:
                @pl.when(k < nch - 1)
                def _():
                    a, d = idx_copies(k + 1, 1 - hk)
                    a.start()
                    d.start()

            def scatter_prev():
                if r >= 2:
                    wait_gather(hk, r - 2, r - 2)
                    fire_scatter(hk, r - 2, r - 2)
                else:
                    wait_gather(1 - hk, r + 6, r + 2)
                    fire_scatter(1 - hk, r + 6, r + 2)

            if r >= 2:
                scatter_prev()
            else:
                pl.when(k > 0)(scatter_prev)

    lh = (nch - 1) % 2
    wait_gather(lh, 6, 2)
    fire_scatter(lh, 6, 2)
    wait_gather(lh, 7, 3)
    fire_scatter(lh, 7, 3)
    for b in range(4):
        wait_scatter(lh, 4 + b, b)
    plsc.subcore_barrier()
    pltpu.sync_copy(acc.at[pl.ds(s * rchunk, rchunk)],
                    out.at[pl.ds(c * NP + s * rchunk, rchunk)])


@functools.partial(
    pl.kernel,
    out_type=jax.ShapeDtypeStruct((SC_CORES * CP, NH), f32),
    mesh=_sc_mesh,
    compiler_params=pltpu.CompilerParams(use_tc_tiling_on_sc=False),
    scratch_types=[
        pltpu.VMEM_SHARED((CP, NH), f32),
        pltpu.VMEM((128, NH), f32),
        pltpu.VMEM((16, 128), jnp.int32),
    ],
)
def _pool_kernel(h, comm2d, zpool, out, accp, hv, idxc):
    c = lax.axis_index("c")
    s = lax.axis_index("s")
    w = c * SC_TILES + s
    pchunk = CP // SC_TILES
    pltpu.sync_copy(zpool.at[pl.ds(s * pchunk, pchunk)],
                    accp.at[pl.ds(s * pchunk, pchunk)])
    nchunks = NPR // 16

    @pl.when(w < nchunks)
    def _():
        pltpu.sync_copy(comm2d.at[pl.ds(w * 16, 16)], idxc)

    plsc.subcore_barrier()

    @pl.when(w < nchunks)
    def _():
        base = w * 16 * 128

        @pl.loop(0, 16)
        def _(j):
            pltpu.sync_copy(h.at[pl.ds(base + j * 128, 128)], hv)
            pltpu.sync_copy(hv, accp.at[idxc.at[j]], add=True)

    plsc.subcore_barrier()
    pltpu.sync_copy(accp.at[pl.ds(s * pchunk, pchunk)],
                    out.at[pl.ds(c * CP + s * pchunk, pchunk)])


_BLK = 512
_NBLK = NP // _BLK


def _dinv_of(degp):
    deg = degp[0] + degp[1] + 1.0
    return lax.rsqrt(deg)


def _dense_body(x_ref, id_ref, degp_ref, w1, b1, w2, b2, w3, b3, wc1, out_ref):
    xb = x_ref[...]
    x1 = jnp.maximum(jnp.dot(xb[:, :7], w1[...],
                             preferred_element_type=f32) + b1[...], 0.0)
    x2 = jnp.maximum(jnp.dot(xb[:, 7:], w2[...],
                             preferred_element_type=f32) + b2[...], 0.0)
    hcat = jnp.concatenate([x1, x2, id_ref[...]], axis=1)
    h = jnp.maximum(jnp.dot(hcat, w3[...],
                            preferred_element_type=f32) + b3[...], 0.0)
    dinv = _dinv_of(degp_ref[...])
    g = jnp.dot(h, wc1[0], preferred_element_type=f32) * dinv[:, None]
    out_ref[...] = g[None]


def _dense_call(xp, idp, deg2, w1, b1, w2, b2, w3, b3, wc1):
    return pl.pallas_call(
        _dense_body,
        grid=(2, _NBLK),
        in_specs=[
            pl.BlockSpec((_BLK, 19), lambda j, i: (i, 0)),
            pl.BlockSpec((_BLK, NID), lambda j, i: (i, 0)),
            pl.BlockSpec((2, _BLK), lambda j, i: (0, i)),
            pl.BlockSpec((7, 32), lambda j, i: (0, 0)),
            pl.BlockSpec((1, 32), lambda j, i: (0, 0)),
            pl.BlockSpec((12, 32), lambda j, i: (0, 0)),
            pl.BlockSpec((1, 32), lambda j, i: (0, 0)),
            pl.BlockSpec((96, 96), lambda j, i: (0, 0)),
            pl.BlockSpec((1, 96), lambda j, i: (0, 0)),
            pl.BlockSpec((1, 96, 32), lambda j, i: (j, 0, 0)),
        ],
        out_specs=pl.BlockSpec((1, _BLK, 32), lambda j, i: (j, i, 0)),
        out_shape=jax.ShapeDtypeStruct((2, NP, 32), f32),
    )(xp, idp, deg2, w1, b1, w2, b2, w3, b3, wc1)


def _post1_body(agg_ref, degp_ref, bc1, wc2, h1_ref, g2_ref):
    ab = agg_ref[...]
    a = jnp.concatenate([ab[0], ab[1]], axis=1)
    dinv = _dinv_of(degp_ref[...])
    h1 = jnp.maximum(a * dinv[:, None] + bc1[...], 0.0)
    h1_ref[...] = h1
    g2_ref[...] = (jnp.dot(h1, wc2[0], preferred_element_type=f32)
                   * dinv[:, None])[None]


def _post1_call(agg1, deg2, bc1, wc2):
    return pl.pallas_call(
        _post1_body,
        grid=(2, _NBLK),
        in_specs=[
            pl.BlockSpec((2, _BLK, 32), lambda j, i: (0, i, 0)),
            pl.BlockSpec((2, _BLK), lambda j, i: (0, i)),
            pl.BlockSpec((1, NH), lambda j, i: (0, 0)),
            pl.BlockSpec((1, NH, 32), lambda j, i: (j, 0, 0)),
        ],
        out_specs=[
            pl.BlockSpec((_BLK, NH), lambda j, i: (i, 0)),
            pl.BlockSpec((1, _BLK, 32), lambda j, i: (j, i, 0)),
        ],
        out_shape=[
            jax.ShapeDtypeStruct((NP, NH), f32),
            jax.ShapeDtypeStruct((2, NP, 32), f32),
        ],
    )(agg1, deg2, bc1, wc2)


def _post2_body(agg_ref, degp_ref, bc2, h2_ref):
    ab = agg_ref[...]
    a = jnp.concatenate([ab[0], ab[1]], axis=1)
    dinv = _dinv_of(degp_ref[...])
    h2_ref[...] = jnp.maximum(a * dinv[:, None] + bc2[...], 0.0)


def _post2_call(agg2, deg2, bc2):
    return pl.pallas_call(
        _post2_body,
        grid=(_NBLK,),
        in_specs=[
            pl.BlockSpec((2, _BLK, 32), lambda i: (0, i, 0)),
            pl.BlockSpec((2, _BLK), lambda i: (0, i)),
            pl.BlockSpec((1, NH), lambda i: (0, 0)),
        ],
        out_specs=pl.BlockSpec((_BLK, NH), lambda i: (i, 0)),
        out_shape=jax.ShapeDtypeStruct((NP, NH), f32),
    )(agg2, deg2, bc2)


def _final_body(p1_ref, p2_ref, cnt_ref, wl1, bl1, wl2, bl2, out_ref):
    p1 = p1_ref[...]
    p2 = p2_ref[...]
    ssum = p1[:CP] + p1[CP:] + p2[:CP] + p2[CP:]
    cntp = cnt_ref[...]
    cnt = jnp.maximum(cntp[0] + cntp[1], 1.0)
    r = ssum / cnt[:, None]
    w = wl1[...]
    wf = w[:NH] + w[NH:]
    z = jnp.maximum(jnp.dot(r, wf, preferred_element_type=f32) + bl1[...], 0.0)
    out_ref[...] = jnp.dot(z, wl2[...], preferred_element_type=f32) + bl2[...]


def _final_call(pools1, pools2, cnt2, wl1, bl1, wl2, bl2):
    return pl.pallas_call(
        _final_body,
        out_shape=jax.ShapeDtypeStruct((CP, 1), f32),
    )(pools1, pools2, cnt2, wl1, bl1, wl2, bl2)


def kernel(x, edge_index, community, id_embeddings, W_emb1, b_emb1, W_emb2,
           b_emb2, W_emb3, b_emb3, W_conv1, b_conv1, W_conv2, b_conv2,
           W_lin1, b_lin1, W_lin2, b_lin2):
    src = edge_index[0]
    dst = edge_index[1]
    epad = jnp.full((EP - E,), N, jnp.int32)
    srcp = jnp.concatenate([src, epad])
    dst2d = jnp.concatenate([dst, epad]).reshape(EPR, 128)
    src2d = jnp.concatenate([srcp, srcp + NP]).reshape(2 * EPR, 128)
    comm2d = jnp.concatenate(
        [community, jnp.full((NP - N,), NCOMM, jnp.int32)]).reshape(NPR, 128)
    xp = jnp.concatenate([x, jnp.zeros((NP - N, x.shape[1]), f32)])
    idp = jnp.concatenate([id_embeddings, jnp.zeros((NP - N, NID), f32)])
    zdeg = jnp.zeros((NP,), f32)
    zcnt = jnp.zeros((CP,), f32)
    zpool = jnp.zeros((CP, NH), f32)

    deg_part, cnt_part = _counts_kernel(dst2d, comm2d, zdeg, zcnt)
    deg2 = deg_part.reshape(2, NP)

    wc1_2 = W_conv1.reshape(96, 2, 32).transpose(1, 0, 2)
    wc2_2 = W_conv2.reshape(NH, 2, 32).transpose(1, 0, 2)
    g1 = _dense_call(xp, idp, deg2, W_emb1, b_emb1.reshape(1, -1), W_emb2,
                     b_emb2.reshape(1, -1), W_emb3, b_emb3.reshape(1, -1),
                     wc1_2)
    agg1 = _agg_kernel(g1.reshape(2 * NP, 32), src2d, dst2d)
    h1, g2 = _post1_call(agg1.reshape(2, NP, 32), deg2,
                         b_conv1.reshape(1, -1), wc2_2)
    pools1 = _pool_kernel(h1, comm2d, zpool)
    agg2 = _agg_kernel(g2.reshape(2 * NP, 32), src2d, dst2d)
    h2 = _post2_call(agg2.reshape(2, NP, 32), deg2, b_conv2.reshape(1, -1))
    pools2 = _pool_kernel(h2, comm2d, zpool)

    res = _final_call(pools1, pools2, cnt_part.reshape(2, CP), W_lin1,
                      b_lin1.reshape(1, -1), W_lin2, b_lin2.reshape(1, 1))
    return res[:NCOMM, 0]
